# baseline (speedup 1.0000x reference)
import functools
import jax
import jax.numpy as jnp
from jax import lax
from jax.experimental import pallas as pl
from jax.experimental.pallas import tpu as pltpu, tpu_sc as plsc

HD = 128
EAD = 16
NU = 10000
NM = 10000
E = 320000

NC = 2
NS = 16
NW = NC * NS
CL = 128
CH = (E + NW * CL - 1) // (NW * CL)
CH = 80
EP = NW * CH * CL
NPAD = 10240
ZR = NPAD // NS

f32 = jnp.float32


def _sigmoid(x):
    return 1.0 / (1.0 + jnp.exp(-x))


def _mm(a, b):
    return jnp.dot(a, b, preferred_element_type=f32)


BN = 1024


def _nodes_pre_body(ux, mx, ueW1, ueb1, ueW2, ueb2, meW1, meb1, meW2, meb2,
                    suWl, smWl, hu, hm, ymx, yux):
    relu = jax.nn.relu
    hu_v = _mm(relu(_mm(ux[...], ueW1[...]) + ueb1[...]), ueW2[...]) + ueb2[...]
    hm_v = _mm(relu(_mm(mx[...], meW1[...]) + meb1[...]), meW2[...]) + meb2[...]
    hu[...] = hu_v
    hm[...] = hm_v
    ymx[...] = _mm(hm_v, suWl[...])
    yux[...] = _mm(hu_v, smWl[...])


def _nodes_pre(ux, mx, ueW1, ueb1, ueW2, ueb2, meW1, meb1, meW2, meb2, suWl, smWl):
    row = pl.BlockSpec((BN, HD), lambda i: (i, 0))
    wfull = lambda shape: pl.BlockSpec(shape, lambda i: (0, 0))
    return pl.pallas_call(
        _nodes_pre_body,
        grid=(NPAD // BN,),
        in_specs=[row, row,
                  wfull((HD, HD)), wfull((1, HD)), wfull((HD, HD)), wfull((1, HD)),
                  wfull((HD, HD)), wfull((1, HD)), wfull((HD, HD)), wfull((1, HD)),
                  wfull((HD, HD)), wfull((HD, HD))],
        out_specs=[row, row, row, row],
        out_shape=[
            jax.ShapeDtypeStruct((NU, HD), f32),
            jax.ShapeDtypeStruct((NM, HD), f32),
            jax.ShapeDtypeStruct((NPAD, HD), f32),
            jax.ShapeDtypeStruct((NPAD, HD), f32),
        ],
    )(ux, mx, ueW1, ueb1, ueW2, ueb2, meW1, meb1, meW2, meb2, suWl, smWl)


def _segsum_body(ymx_h, yux_h, srcu_h, dstu_h, srcm_h, dstm_h, zeros_h,
                 zeros1_h, ones_h,
                 outu_h, outm_h, cntu_h, cntm_h,
                 sidx0, sidx1, didx_v, buf0, buf1, ones_v, tab_s, cnt_s,
                 sem0, sem1, semw0, semw1, semc, semi0, semi1):
    cid = lax.axis_index("c")
    sid = lax.axis_index("s")
    wid = cid * NS + sid
    pltpu.sync_copy(ones_h, ones_v)

    def do_relation(tab_hbm, src_h, dst_h, out_h, cnt_h):
        pltpu.sync_copy(zeros_h.at[pl.ds(sid * ZR, ZR)], tab_s.at[pl.ds(sid * ZR, ZR)])
        pltpu.sync_copy(zeros1_h.at[pl.ds(sid * ZR, ZR)], cnt_s.at[pl.ds(sid * ZR, ZR)])
        pltpu.sync_copy(dst_h.at[wid], didx_v)
        plsc.subcore_barrier()

        pltpu.sync_copy(src_h.at[wid, 0], sidx0)
        pltpu.async_copy(tab_hbm.at[sidx0], buf0, sem0)
        pltpu.sync_copy(src_h.at[wid, 1], sidx1)
        pltpu.async_copy(tab_hbm.at[sidx1], buf1, sem1)

        def body(g, carry):
            j0 = 2 * g
            j1 = 2 * g + 1
            j2 = lax.rem(2 * g + 2, CH)
            j3 = lax.rem(2 * g + 3, CH)
            pltpu.make_async_copy(tab_hbm.at[sidx0], buf0, sem0).wait()
            pltpu.async_copy(buf0, tab_s.at[didx_v.at[j0]], semw0, add=True)
            pltpu.async_copy(ones_v, cnt_s.at[didx_v.at[j0]], semc, add=True)
            pltpu.async_copy(src_h.at[wid, j2], sidx0, semi0)
            pltpu.make_async_copy(tab_hbm.at[sidx1], buf1, sem1).wait()
            pltpu.async_copy(buf1, tab_s.at[didx_v.at[j1]], semw1, add=True)
            pltpu.async_copy(ones_v, cnt_s.at[didx_v.at[j1]], semc, add=True)
            pltpu.async_copy(src_h.at[wid, j3], sidx1, semi1)
            pltpu.make_async_copy(buf0, tab_s.at[didx_v.at[j0]], semw0).wait()
            pltpu.make_async_copy(src_h.at[wid, j2], sidx0, semi0).wait()
            pltpu.async_copy(tab_hbm.at[sidx0], buf0, sem0)
            pltpu.make_async_copy(buf1, tab_s.at[didx_v.at[j1]], semw1).wait()
            pltpu.make_async_copy(src_h.at[wid, j3], sidx1, semi1).wait()
            pltpu.async_copy(tab_hbm.at[sidx1], buf1, sem1)
            return carry

        lax.fori_loop(0, CH // 2, body, 0)
        pltpu.make_async_copy(tab_hbm.at[sidx0], buf0, sem0).wait()
        pltpu.make_async_copy(tab_hbm.at[sidx1], buf1, sem1).wait()

        def drain(j, carry):
            pltpu.make_async_copy(ones_v, cnt_s.at[didx_v.at[0]], semc).wait()
            return carry

        lax.fori_loop(0, CH, drain, 0)
        plsc.subcore_barrier()
        pltpu.sync_copy(tab_s.at[pl.ds(sid * ZR, ZR)],
                        out_h.at[cid, pl.ds(sid * ZR, ZR)])
        pltpu.sync_copy(cnt_s.at[pl.ds(sid * ZR, ZR)],
                        cnt_h.at[cid, pl.ds(sid * ZR, ZR)])
        plsc.subcore_barrier()

    do_relation(ymx_h, srcu_h, dstu_h, outu_h, cntu_h)
    do_relation(yux_h, srcm_h, dstm_h, outm_h, cntm_h)


def _segsum(ymx, yux, srcu, dstu, srcm, dstm, zeros_tab, zeros_col, ones_cl):
    mesh = plsc.VectorSubcoreMesh(core_axis_name="c", subcore_axis_name="s")
    return pl.kernel(
        _segsum_body,
        out_type=[
            jax.ShapeDtypeStruct((NC, NPAD, HD), f32),
            jax.ShapeDtypeStruct((NC, NPAD, HD), f32),
            jax.ShapeDtypeStruct((NC, NPAD), f32),
            jax.ShapeDtypeStruct((NC, NPAD), f32),
        ],
        mesh=mesh,
        scratch_types=[
            pltpu.VMEM((CL,), jnp.int32),
            pltpu.VMEM((CL,), jnp.int32),
            pltpu.VMEM((CH, CL), jnp.int32),
            pltpu.VMEM((CL, HD), f32),
            pltpu.VMEM((CL, HD), f32),
            pltpu.VMEM((CL,), f32),
            pltpu.VMEM_SHARED((NPAD, HD), f32),
            pltpu.VMEM_SHARED((NPAD,), f32),
            pltpu.SemaphoreType.DMA,
            pltpu.SemaphoreType.DMA,
            pltpu.SemaphoreType.DMA,
            pltpu.SemaphoreType.DMA,
            pltpu.SemaphoreType.DMA,
            pltpu.SemaphoreType.DMA,
            pltpu.SemaphoreType.DMA,
        ],
    )(ymx, yux, srcu, dstu, srcm, dstm, zeros_tab, zeros_col, ones_cl)


def _nodes_post_body(hu0, hm0, ptu, ptm, ctu, ctm,
                     suWr, subl, suP1, supb1, supW2, supb2,
                     smWr, smbl, smP1, smpb1, smpW2, smpb2,
                     iW1, ib1, iW2p, ib2p, tW1, tb1, tW2, tb2,
                     npW, npb, W1a, W1b, eeW2, eeb2, W1c, mlpb1,
                     pu, pm, w1ce, be):
    relu = jax.nn.relu
    hu0v = hu0[...]
    hm0v = hm0[...]

    aggu = ptu[0] + ptu[1]
    cntu = jnp.maximum(ctu[:, 0:1] + ctu[:, 1:2], 1.0)
    u_conv = aggu / cntu + subl[...] + _mm(hu0v, suWr[...])
    h_u_rel = _mm(relu(_mm(u_conv, suP1[...]) + supb1[...]), supW2[...]) + supb2[...]

    aggm = ptm[0] + ptm[1]
    cntm = jnp.maximum(ctm[:, 0:1] + ctm[:, 1:2], 1.0)
    m_conv = aggm / cntm + smbl[...] + _mm(hm0v, smWr[...])
    h_m_rel = _mm(relu(_mm(m_conv, smP1[...]) + smpb1[...]), smpW2[...]) + smpb2[...]

    hu = hu0v + h_u_rel
    hm = hm0v + h_m_rel

    def intens(h):
        imp = _sigmoid(_mm(relu(_mm(h, iW1[...]) + ib1[...]), iW2p[...]) + ib2p[...])[:, 0:1]
        t = _mm(relu(_mm(h, tW1[...]) + tb1[...]), tW2[...]) + tb2[...]
        return h + t * imp

    hu = _mm(intens(hu), npW[...]) + npb[...]
    hm = _mm(intens(hm), npW[...]) + npb[...]

    pu[...] = _mm(hu, W1a[...])
    pm[...] = _mm(hm, W1b[...])
    w1ce[...] = _mm(eeW2[...], W1c[...])
    be[...] = _mm(eeb2[...], W1c[...]) + mlpb1[...]


def _nodes_post(*args):
    row = pl.BlockSpec((BN, HD), lambda i: (i, 0))
    wfull = lambda shape: pl.BlockSpec(shape, lambda i: (0,) * len(shape))
    pt = pl.BlockSpec((NC, BN, HD), lambda i: (0, i, 0))
    ct = pl.BlockSpec((BN, NC), lambda i: (i, 0))
    w = wfull((HD, HD))
    b = wfull((1, HD))
    return pl.pallas_call(
        _nodes_post_body,
        grid=(NPAD // BN,),
        in_specs=[row, row, pt, pt, ct, ct,
                  w, b, w, b, w, b,
                  w, b, w, b, w, b,
                  wfull((HD, HD // 2)), wfull((1, HD // 2)), wfull((HD // 2, HD)), b,
                  w, b, w, b,
                  w, b, w, w, w, b, w, b],
        out_specs=[row, row, w, b],
        out_shape=[
            jax.ShapeDtypeStruct((NPAD, HD), f32),
            jax.ShapeDtypeStruct((NPAD, HD), f32),
            jax.ShapeDtypeStruct((HD, HD), f32),
            jax.ShapeDtypeStruct((1, HD), f32),
        ],
    )(*args)


def _vadd_chunk(ba, bb):
    def row(r, carry):
        for k in range(HD // 16):
            sl = pl.ds(k * 16, 16)
            ba[r, sl] = ba[r, sl] + bb[r, sl]
        return carry

    lax.fori_loop(0, CL, row, 0)


def _gather_body(pu_h, pm_h, src_h, dst_h, gs_h,
                 sidx_v, didx_v, bufa0, bufa1, bufb0, bufb1,
                 sema0, sema1, semb0, semb1):
    cid = lax.axis_index("c")
    sid = lax.axis_index("s")
    wid = cid * NS + sid
    base = wid * (CH * CL)
    pltpu.sync_copy(src_h.at[wid], sidx_v)
    pltpu.sync_copy(dst_h.at[wid], didx_v)

    pltpu.async_copy(pu_h.at[sidx_v.at[0]], bufa0, sema0)
    pltpu.async_copy(pm_h.at[didx_v.at[0]], bufb0, semb0)

    def body(g, carry):
        j0 = 2 * g
        j1 = 2 * g + 1
        j2 = lax.rem(2 * g + 2, CH)
        pltpu.async_copy(pu_h.at[sidx_v.at[j1]], bufa1, sema1)
        pltpu.async_copy(pm_h.at[didx_v.at[j1]], bufb1, semb1)
        pltpu.make_async_copy(pu_h.at[sidx_v.at[j0]], bufa0, sema0).wait()
        pltpu.make_async_copy(pm_h.at[didx_v.at[j0]], bufb0, semb0).wait()
        _vadd_chunk(bufa0, bufb0)
        pltpu.sync_copy(bufa0, gs_h.at[pl.ds(base + j0 * CL, CL)])
        pltpu.async_copy(pu_h.at[sidx_v.at[j2]], bufa0, sema0)
        pltpu.async_copy(pm_h.at[didx_v.at[j2]], bufb0, semb0)
        pltpu.make_async_copy(pu_h.at[sidx_v.at[j1]], bufa1, sema1).wait()
        pltpu.make_async_copy(pm_h.at[didx_v.at[j1]], bufb1, semb1).wait()
        _vadd_chunk(bufa1, bufb1)
        pltpu.sync_copy(bufa1, gs_h.at[pl.ds(base + j1 * CL, CL)])
        return carry

    lax.fori_loop(0, CH // 2, body, 0)
    pltpu.make_async_copy(pu_h.at[sidx_v.at[0]], bufa0, sema0).wait()
    pltpu.make_async_copy(pm_h.at[didx_v.at[0]], bufb0, semb0).wait()


def _gathers(pu, pm, srce, dste):
    mesh = plsc.VectorSubcoreMesh(core_axis_name="c", subcore_axis_name="s")
    return pl.kernel(
        _gather_body,
        out_type=jax.ShapeDtypeStruct((EP, HD), f32),
        mesh=mesh,
        scratch_types=[
            pltpu.VMEM((CH, CL), jnp.int32),
            pltpu.VMEM((CH, CL), jnp.int32),
            pltpu.VMEM((CL, HD), f32),
            pltpu.VMEM((CL, HD), f32),
            pltpu.VMEM((CL, HD), f32),
            pltpu.VMEM((CL, HD), f32),
            pltpu.SemaphoreType.DMA,
            pltpu.SemaphoreType.DMA,
            pltpu.SemaphoreType.DMA,
            pltpu.SemaphoreType.DMA,
        ],
    )(pu, pm, srce, dste)


BK = 2560


def _edges_body(gs, eat, eeW1, eeb1, w1ce, be, W2, b2, W3, b3t, out):
    relu = jax.nn.relu
    ea_w = lax.dot_general(eat[...], eeW1[...], (((0,), (0,)), ((), ())),
                           preferred_element_type=f32)
    h0 = relu(ea_w + eeb1[...])
    h1 = relu(gs[...] + _mm(h0, w1ce[...]) + be[...])
    h2 = relu(_mm(h1, W2[...]) + b2[...])
    out[...] = lax.dot_general(W3[...], h2, (((0,), (1,)), ((), ())),
                               preferred_element_type=f32) + b3t[...]


def _edges(gs, eat, eeW1, eeb1, w1ce, be, W2, b2, W3, b3t):
    nblk = E // BK
    full = lambda shape: pl.BlockSpec(shape, lambda i: (0, 0))
    return pl.pallas_call(
        _edges_body,
        grid=(nblk,),
        in_specs=[
            pl.BlockSpec((BK, HD), lambda i: (i, 0)),
            pl.BlockSpec((EAD, BK), lambda i: (0, i)),
            full((EAD, HD)),
            full((1, HD)),
            full((HD, HD)),
            full((1, HD)),
            full((HD, HD // 2)),
            full((1, HD // 2)),
            full((HD // 2, 2)),
            full((2, 1)),
        ],
        out_specs=pl.BlockSpec((2, BK), lambda i: (0, i)),
        out_shape=jax.ShapeDtypeStruct((2, E), f32),
    )(gs, eat, eeW1, eeb1, w1ce, be, W2, b2, W3, b3t)


def _prep_idx(idx):
    pad = NU + (jnp.arange(EP - E, dtype=jnp.int32) % (NPAD - NU))
    full = jnp.concatenate([idx.astype(jnp.int32), pad])
    return full.reshape(NW, CH, CL)


def kernel(user_x, merchant_x, edge_index_ut, edge_index_mr, src_idx, dst_idx,
           edge_attr, params):
    p = params
    r1 = lambda b: b.reshape(1, -1).astype(f32)

    srcu = _prep_idx(edge_index_mr[0])
    dstu = _prep_idx(edge_index_mr[1])
    srcm = _prep_idx(edge_index_ut[0])
    dstm = _prep_idx(edge_index_ut[1])
    srce = _prep_idx(src_idx)
    dste = _prep_idx(dst_idx)
    zeros_tab = jnp.zeros((NPAD, HD), f32)
    zeros_col = jnp.zeros((NPAD,), f32)
    ones_cl = jnp.ones((CL,), f32)

    hu0, hm0, ymx, yux = _nodes_pre(
        user_x, merchant_x,
        p['ue_W1'], r1(p['ue_b1']), p['ue_W2'], r1(p['ue_b2']),
        p['me_W1'], r1(p['me_b1']), p['me_W2'], r1(p['me_b2']),
        p['su_Wl'], p['sm_Wl'])

    ptu, ptm, ctu, ctm = _segsum(ymx, yux, srcu, dstu, srcm, dstm,
                                 zeros_tab, zeros_col, ones_cl)

    iW2p = jnp.zeros((HD // 2, HD), f32).at[:, 0].set(p['int_iW2'][:, 0])
    ib2p = jnp.zeros((1, HD), f32).at[0, 0].set(p['int_ib2'][0])

    pu, pm, w1ce, be = _nodes_post(
        hu0, hm0, ptu, ptm, jnp.transpose(ctu), jnp.transpose(ctm),
        p['su_Wr'], r1(p['su_bl']), p['su_pW1'][HD:], r1(p['su_pb1']),
        p['su_pW2'], r1(p['su_pb2']),
        p['sm_Wr'], r1(p['sm_bl']), p['sm_pW1'][:HD], r1(p['sm_pb1']),
        p['sm_pW2'], r1(p['sm_pb2']),
        p['int_iW1'], r1(p['int_ib1']), iW2p, ib2p,
        p['int_tW1'], r1(p['int_tb1']), p['int_tW2'], r1(p['int_tb2']),
        p['np_W'], r1(p['np_b']),
        p['mlp_W1'][:HD], p['mlp_W1'][HD:2 * HD],
        p['ee_W2'], r1(p['ee_b2']), p['mlp_W1'][2 * HD:], r1(p['mlp_b1']))

    gs = _gathers(pu, pm, srce, dste)

    logits_t = _edges(gs, jnp.transpose(edge_attr),
                      p['ee_W1'], r1(p['ee_b1']), w1ce, be,
                      p['mlp_W2'], r1(p['mlp_b2']), p['mlp_W3'],
                      p['mlp_b3'].reshape(2, 1).astype(f32))
    return jnp.transpose(logits_t)


# speedup vs baseline: 1.0640x; 1.0640x over previous
import functools
import jax
import jax.numpy as jnp
from jax import lax
from jax.experimental import pallas as pl
from jax.experimental.pallas import tpu as pltpu, tpu_sc as plsc

HD = 128
EAD = 16
NU = 10000
NM = 10000
E = 320000

NC = 2
NS = 16
NW = NC * NS
CL = 128
CH = (E + NW * CL - 1) // (NW * CL)
CH = 80
EP = NW * CH * CL
NPAD = 10240
ZR = NPAD // NS

f32 = jnp.float32


def _sigmoid(x):
    return 1.0 / (1.0 + jnp.exp(-x))


def _mm(a, b):
    return jnp.dot(a, b, preferred_element_type=f32)


BN = 1024


def _nodes_pre_body(ux, mx, ueW1, ueb1, ueW2, ueb2, meW1, meb1, meW2, meb2,
                    suWl, smWl, hu, hm, ymx, yux):
    relu = jax.nn.relu
    hu_v = _mm(relu(_mm(ux[...], ueW1[...]) + ueb1[...]), ueW2[...]) + ueb2[...]
    hm_v = _mm(relu(_mm(mx[...], meW1[...]) + meb1[...]), meW2[...]) + meb2[...]
    hu[...] = hu_v
    hm[...] = hm_v
    ymx[...] = _mm(hm_v, suWl[...])
    yux[...] = _mm(hu_v, smWl[...])


def _nodes_pre(ux, mx, ueW1, ueb1, ueW2, ueb2, meW1, meb1, meW2, meb2, suWl, smWl):
    row = pl.BlockSpec((BN, HD), lambda i: (i, 0))
    wfull = lambda shape: pl.BlockSpec(shape, lambda i: (0, 0))
    return pl.pallas_call(
        _nodes_pre_body,
        grid=(NPAD // BN,),
        in_specs=[row, row,
                  wfull((HD, HD)), wfull((1, HD)), wfull((HD, HD)), wfull((1, HD)),
                  wfull((HD, HD)), wfull((1, HD)), wfull((HD, HD)), wfull((1, HD)),
                  wfull((HD, HD)), wfull((HD, HD))],
        out_specs=[row, row, row, row],
        out_shape=[
            jax.ShapeDtypeStruct((NU, HD), f32),
            jax.ShapeDtypeStruct((NM, HD), f32),
            jax.ShapeDtypeStruct((NPAD, HD), f32),
            jax.ShapeDtypeStruct((NPAD, HD), f32),
        ],
    )(ux, mx, ueW1, ueb1, ueW2, ueb2, meW1, meb1, meW2, meb2, suWl, smWl)


def _segsum_body(ymx_h, yux_h, srcu_h, dstu_h, srcm_h, dstm_h, zeros_h,
                 zeros1_h, ones_h,
                 outu_h, outm_h, cntu_h, cntm_h,
                 sidx0, sidx1, didx_v, buf0, buf1, ones_v, tab_s, cnt_s,
                 sem0, sem1, semw0, semw1, semc, semi0, semi1):
    cid = lax.axis_index("c")
    sid = lax.axis_index("s")
    wid = cid * NS + sid
    pltpu.sync_copy(ones_h, ones_v)

    def do_relation(tab_hbm, src_h, dst_h, out_h, cnt_h):
        pltpu.sync_copy(zeros_h.at[pl.ds(sid * ZR, ZR)], tab_s.at[pl.ds(sid * ZR, ZR)])
        pltpu.sync_copy(zeros1_h.at[pl.ds(sid * ZR, ZR)], cnt_s.at[pl.ds(sid * ZR, ZR)])
        pltpu.sync_copy(dst_h.at[wid], didx_v)
        plsc.subcore_barrier()

        pltpu.sync_copy(src_h.at[wid, 0], sidx0)
        pltpu.async_copy(tab_hbm.at[sidx0], buf0, sem0)
        pltpu.sync_copy(src_h.at[wid, 1], sidx1)
        pltpu.async_copy(tab_hbm.at[sidx1], buf1, sem1)

        def body(g, carry):
            j0 = 2 * g
            j1 = 2 * g + 1
            j2 = lax.rem(2 * g + 2, CH)
            j3 = lax.rem(2 * g + 3, CH)
            pltpu.make_async_copy(tab_hbm.at[sidx0], buf0, sem0).wait()
            pltpu.async_copy(buf0, tab_s.at[didx_v.at[j0]], semw0, add=True)
            pltpu.async_copy(ones_v, cnt_s.at[didx_v.at[j0]], semc, add=True)
            pltpu.async_copy(src_h.at[wid, j2], sidx0, semi0)
            pltpu.make_async_copy(tab_hbm.at[sidx1], buf1, sem1).wait()
            pltpu.async_copy(buf1, tab_s.at[didx_v.at[j1]], semw1, add=True)
            pltpu.async_copy(ones_v, cnt_s.at[didx_v.at[j1]], semc, add=True)
            pltpu.async_copy(src_h.at[wid, j3], sidx1, semi1)
            pltpu.make_async_copy(buf0, tab_s.at[didx_v.at[j0]], semw0).wait()
            pltpu.make_async_copy(src_h.at[wid, j2], sidx0, semi0).wait()
            pltpu.async_copy(tab_hbm.at[sidx0], buf0, sem0)
            pltpu.make_async_copy(buf1, tab_s.at[didx_v.at[j1]], semw1).wait()
            pltpu.make_async_copy(src_h.at[wid, j3], sidx1, semi1).wait()
            pltpu.async_copy(tab_hbm.at[sidx1], buf1, sem1)
            return carry

        lax.fori_loop(0, CH // 2, body, 0)
        pltpu.make_async_copy(tab_hbm.at[sidx0], buf0, sem0).wait()
        pltpu.make_async_copy(tab_hbm.at[sidx1], buf1, sem1).wait()

        def drain(j, carry):
            pltpu.make_async_copy(ones_v, cnt_s.at[didx_v.at[0]], semc).wait()
            return carry

        lax.fori_loop(0, CH, drain, 0)
        plsc.subcore_barrier()
        pltpu.sync_copy(tab_s.at[pl.ds(sid * ZR, ZR)],
                        out_h.at[cid, pl.ds(sid * ZR, ZR)])
        pltpu.sync_copy(cnt_s.at[pl.ds(sid * ZR, ZR)],
                        cnt_h.at[cid, pl.ds(sid * ZR, ZR)])
        plsc.subcore_barrier()

    do_relation(ymx_h, srcu_h, dstu_h, outu_h, cntu_h)
    do_relation(yux_h, srcm_h, dstm_h, outm_h, cntm_h)


def _segsum(ymx, yux, srcu, dstu, srcm, dstm, zeros_tab, zeros_col, ones_cl):
    mesh = plsc.VectorSubcoreMesh(core_axis_name="c", subcore_axis_name="s")
    return pl.kernel(
        _segsum_body,
        out_type=[
            jax.ShapeDtypeStruct((NC, NPAD, HD), f32),
            jax.ShapeDtypeStruct((NC, NPAD, HD), f32),
            jax.ShapeDtypeStruct((NC, NPAD), f32),
            jax.ShapeDtypeStruct((NC, NPAD), f32),
        ],
        mesh=mesh,
        scratch_types=[
            pltpu.VMEM((CL,), jnp.int32),
            pltpu.VMEM((CL,), jnp.int32),
            pltpu.VMEM((CH, CL), jnp.int32),
            pltpu.VMEM((CL, HD), f32),
            pltpu.VMEM((CL, HD), f32),
            pltpu.VMEM((CL,), f32),
            pltpu.VMEM_SHARED((NPAD, HD), f32),
            pltpu.VMEM_SHARED((NPAD,), f32),
            pltpu.SemaphoreType.DMA,
            pltpu.SemaphoreType.DMA,
            pltpu.SemaphoreType.DMA,
            pltpu.SemaphoreType.DMA,
            pltpu.SemaphoreType.DMA,
            pltpu.SemaphoreType.DMA,
            pltpu.SemaphoreType.DMA,
        ],
    )(ymx, yux, srcu, dstu, srcm, dstm, zeros_tab, zeros_col, ones_cl)


def _nodes_post_body(hu0, hm0, ptu, ptm, ctu, ctm,
                     suWr, subl, suP1, supb1, supW2, supb2,
                     smWr, smbl, smP1, smpb1, smpW2, smpb2,
                     iW1, ib1, iW2p, ib2p, tW1, tb1, tW2, tb2,
                     npW, npb, W1a, W1b, eeW2, eeb2, W1c, mlpb1,
                     pu, pm, w1ce, be):
    relu = jax.nn.relu
    hu0v = hu0[...]
    hm0v = hm0[...]

    aggu = ptu[0] + ptu[1]
    cntu = jnp.maximum(ctu[:, 0:1] + ctu[:, 1:2], 1.0)
    u_conv = aggu / cntu + subl[...] + _mm(hu0v, suWr[...])
    h_u_rel = _mm(relu(_mm(u_conv, suP1[...]) + supb1[...]), supW2[...]) + supb2[...]

    aggm = ptm[0] + ptm[1]
    cntm = jnp.maximum(ctm[:, 0:1] + ctm[:, 1:2], 1.0)
    m_conv = aggm / cntm + smbl[...] + _mm(hm0v, smWr[...])
    h_m_rel = _mm(relu(_mm(m_conv, smP1[...]) + smpb1[...]), smpW2[...]) + smpb2[...]

    hu = hu0v + h_u_rel
    hm = hm0v + h_m_rel

    def intens(h):
        imp = _sigmoid(_mm(relu(_mm(h, iW1[...]) + ib1[...]), iW2p[...]) + ib2p[...])[:, 0:1]
        t = _mm(relu(_mm(h, tW1[...]) + tb1[...]), tW2[...]) + tb2[...]
        return h + t * imp

    hu = _mm(intens(hu), npW[...]) + npb[...]
    hm = _mm(intens(hm), npW[...]) + npb[...]

    pu[...] = _mm(hu, W1a[...])
    pm[...] = _mm(hm, W1b[...])
    w1ce[...] = _mm(eeW2[...], W1c[...])
    be[...] = _mm(eeb2[...], W1c[...]) + mlpb1[...]


def _nodes_post(*args):
    row = pl.BlockSpec((BN, HD), lambda i: (i, 0))
    wfull = lambda shape: pl.BlockSpec(shape, lambda i: (0,) * len(shape))
    pt = pl.BlockSpec((NC, BN, HD), lambda i: (0, i, 0))
    ct = pl.BlockSpec((BN, NC), lambda i: (i, 0))
    w = wfull((HD, HD))
    b = wfull((1, HD))
    return pl.pallas_call(
        _nodes_post_body,
        grid=(NPAD // BN,),
        in_specs=[row, row, pt, pt, ct, ct,
                  w, b, w, b, w, b,
                  w, b, w, b, w, b,
                  wfull((HD, HD // 2)), wfull((1, HD // 2)), wfull((HD // 2, HD)), b,
                  w, b, w, b,
                  w, b, w, w, w, b, w, b],
        out_specs=[row, row, w, b],
        out_shape=[
            jax.ShapeDtypeStruct((NPAD, HD), f32),
            jax.ShapeDtypeStruct((NPAD, HD), f32),
            jax.ShapeDtypeStruct((HD, HD), f32),
            jax.ShapeDtypeStruct((1, HD), f32),
        ],
    )(*args)


def _vadd_chunk(ba, bb):
    def row(r, carry):
        for k in range(HD // 16):
            sl = pl.ds(k * 16, 16)
            ba[r, sl] = ba[r, sl] + bb[r, sl]
        return carry

    lax.fori_loop(0, CL, row, 0)


E2 = E // 2
CHH = 40
EPH = NW * CHH * CL


def _gather_body(pu_h, pm_h, src_h, dst_h, gs_h,
                 sidx_v, didx_v, bufa0, bufa1, bufb0, bufb1,
                 sema0, sema1, semb0, semb1):
    cid = lax.axis_index("c")
    sid = lax.axis_index("s")
    wid = cid * NS + sid
    base = wid * (CHH * CL)
    pltpu.sync_copy(src_h.at[wid], sidx_v)
    pltpu.sync_copy(dst_h.at[wid], didx_v)

    pltpu.async_copy(pu_h.at[sidx_v.at[0]], bufa0, sema0)
    pltpu.async_copy(pm_h.at[didx_v.at[0]], bufb0, semb0)

    def body(g, carry):
        j0 = 2 * g
        j1 = 2 * g + 1
        j2 = lax.rem(2 * g + 2, CHH)
        pltpu.async_copy(pu_h.at[sidx_v.at[j1]], bufa1, sema1)
        pltpu.async_copy(pm_h.at[didx_v.at[j1]], bufb1, semb1)
        pltpu.make_async_copy(pu_h.at[sidx_v.at[j0]], bufa0, sema0).wait()
        pltpu.make_async_copy(pm_h.at[didx_v.at[j0]], bufb0, semb0).wait()
        _vadd_chunk(bufa0, bufb0)
        pltpu.sync_copy(bufa0, gs_h.at[pl.ds(base + j0 * CL, CL)])
        pltpu.async_copy(pu_h.at[sidx_v.at[j2]], bufa0, sema0)
        pltpu.async_copy(pm_h.at[didx_v.at[j2]], bufb0, semb0)
        pltpu.make_async_copy(pu_h.at[sidx_v.at[j1]], bufa1, sema1).wait()
        pltpu.make_async_copy(pm_h.at[didx_v.at[j1]], bufb1, semb1).wait()
        _vadd_chunk(bufa1, bufb1)
        pltpu.sync_copy(bufa1, gs_h.at[pl.ds(base + j1 * CL, CL)])
        return carry

    lax.fori_loop(0, CHH // 2, body, 0)
    pltpu.make_async_copy(pu_h.at[sidx_v.at[0]], bufa0, sema0).wait()
    pltpu.make_async_copy(pm_h.at[didx_v.at[0]], bufb0, semb0).wait()


def _gathers(pu, pm, srce, dste):
    mesh = plsc.VectorSubcoreMesh(core_axis_name="c", subcore_axis_name="s")
    return pl.kernel(
        _gather_body,
        out_type=jax.ShapeDtypeStruct((EPH, HD), f32),
        mesh=mesh,
        scratch_types=[
            pltpu.VMEM((CHH, CL), jnp.int32),
            pltpu.VMEM((CHH, CL), jnp.int32),
            pltpu.VMEM((CL, HD), f32),
            pltpu.VMEM((CL, HD), f32),
            pltpu.VMEM((CL, HD), f32),
            pltpu.VMEM((CL, HD), f32),
            pltpu.SemaphoreType.DMA,
            pltpu.SemaphoreType.DMA,
            pltpu.SemaphoreType.DMA,
            pltpu.SemaphoreType.DMA,
        ],
    )(pu, pm, srce, dste)


BK = 3200


def _edges_body(gs, eat, eeW1, eeb1, w1ce, be, W2, b2, W3, b3t, out):
    relu = jax.nn.relu
    ea_w = lax.dot_general(eat[...], eeW1[...], (((0,), (0,)), ((), ())),
                           preferred_element_type=f32)
    h0 = relu(ea_w + eeb1[...])
    h1 = relu(gs[...] + _mm(h0, w1ce[...]) + be[...])
    h2 = relu(_mm(h1, W2[...]) + b2[...])
    out[...] = lax.dot_general(W3[...], h2, (((0,), (1,)), ((), ())),
                               preferred_element_type=f32) + b3t[...]


def _edges(gs, eat, eeW1, eeb1, w1ce, be, W2, b2, W3, b3t, half):
    nblk = E2 // BK
    off = half * nblk
    full = lambda shape: pl.BlockSpec(shape, lambda i: (0, 0))
    return pl.pallas_call(
        _edges_body,
        grid=(nblk,),
        in_specs=[
            pl.BlockSpec((BK, HD), lambda i: (i, 0)),
            pl.BlockSpec((EAD, BK), lambda i: (0, i + off)),
            full((EAD, HD)),
            full((1, HD)),
            full((HD, HD)),
            full((1, HD)),
            full((HD, HD // 2)),
            full((1, HD // 2)),
            full((HD // 2, 2)),
            full((2, 1)),
        ],
        out_specs=pl.BlockSpec((2, BK), lambda i: (0, i)),
        out_shape=jax.ShapeDtypeStruct((2, E2), f32),
    )(gs, eat, eeW1, eeb1, w1ce, be, W2, b2, W3, b3t)


def _prep_idx(idx):
    pad = NU + (jnp.arange(EP - E, dtype=jnp.int32) % (NPAD - NU))
    full = jnp.concatenate([idx.astype(jnp.int32), pad])
    return full.reshape(NW, CH, CL)


def _prep_idx_half(idx):
    pad = NU + (jnp.arange(EPH - E2, dtype=jnp.int32) % (NPAD - NU))
    full = jnp.concatenate([idx.astype(jnp.int32), pad])
    return full.reshape(NW, CHH, CL)


def kernel(user_x, merchant_x, edge_index_ut, edge_index_mr, src_idx, dst_idx,
           edge_attr, params):
    p = params
    r1 = lambda b: b.reshape(1, -1).astype(f32)

    srcu = _prep_idx(edge_index_mr[0])
    dstu = _prep_idx(edge_index_mr[1])
    srcm = _prep_idx(edge_index_ut[0])
    dstm = _prep_idx(edge_index_ut[1])
    srce1 = _prep_idx_half(src_idx[:E2])
    srce2 = _prep_idx_half(src_idx[E2:])
    dste1 = _prep_idx_half(dst_idx[:E2])
    dste2 = _prep_idx_half(dst_idx[E2:])
    zeros_tab = jnp.zeros((NPAD, HD), f32)
    zeros_col = jnp.zeros((NPAD,), f32)
    ones_cl = jnp.ones((CL,), f32)

    hu0, hm0, ymx, yux = _nodes_pre(
        user_x, merchant_x,
        p['ue_W1'], r1(p['ue_b1']), p['ue_W2'], r1(p['ue_b2']),
        p['me_W1'], r1(p['me_b1']), p['me_W2'], r1(p['me_b2']),
        p['su_Wl'], p['sm_Wl'])

    ptu, ptm, ctu, ctm = _segsum(ymx, yux, srcu, dstu, srcm, dstm,
                                 zeros_tab, zeros_col, ones_cl)

    iW2p = jnp.zeros((HD // 2, HD), f32).at[:, 0].set(p['int_iW2'][:, 0])
    ib2p = jnp.zeros((1, HD), f32).at[0, 0].set(p['int_ib2'][0])

    pu, pm, w1ce, be = _nodes_post(
        hu0, hm0, ptu, ptm, jnp.transpose(ctu), jnp.transpose(ctm),
        p['su_Wr'], r1(p['su_bl']), p['su_pW1'][HD:], r1(p['su_pb1']),
        p['su_pW2'], r1(p['su_pb2']),
        p['sm_Wr'], r1(p['sm_bl']), p['sm_pW1'][:HD], r1(p['sm_pb1']),
        p['sm_pW2'], r1(p['sm_pb2']),
        p['int_iW1'], r1(p['int_ib1']), iW2p, ib2p,
        p['int_tW1'], r1(p['int_tb1']), p['int_tW2'], r1(p['int_tb2']),
        p['np_W'], r1(p['np_b']),
        p['mlp_W1'][:HD], p['mlp_W1'][HD:2 * HD],
        p['ee_W2'], r1(p['ee_b2']), p['mlp_W1'][2 * HD:], r1(p['mlp_b1']))

    eat = jnp.transpose(edge_attr)
    b3t = p['mlp_b3'].reshape(2, 1).astype(f32)
    ew = (p['ee_W1'], r1(p['ee_b1']), w1ce, be,
          p['mlp_W2'], r1(p['mlp_b2']), p['mlp_W3'], b3t)

    gs1 = _gathers(pu, pm, srce1, dste1)
    gs2 = _gathers(pu, pm, srce2, dste2)
    lt1 = _edges(gs1, eat, *ew, half=0)
    lt2 = _edges(gs2, eat, *ew, half=1)
    return jnp.transpose(jnp.concatenate([lt1, lt2], axis=1))
